# baseline (speedup 1.0000x reference)
import functools

import jax
import jax.numpy as jnp
from jax import lax
from jax.experimental import pallas as pl
from jax.experimental.pallas import tpu as pltpu
from jax.experimental.pallas import tpu_sc as plsc

N = 10000
H = 128
NC = 2
NS = 16
CHUNK = 128
N_ACC = 10240
ZROWS = N_ACC // NS
BR = 2000


def _mesh():
    return plsc.VectorSubcoreMesh(
        core_axis_name="c", subcore_axis_name="s", num_cores=NC, num_subcores=NS
    )


def _make_scatter(total_chunks, cpt0):
    cpt1 = total_chunks // NS - cpt0

    @functools.partial(
        pl.kernel,
        mesh=_mesh(),
        out_type=jax.ShapeDtypeStruct((NC * N_ACC, H), jnp.float32),
        scratch_types=[
            pltpu.VMEM((CHUNK,), jnp.int32),
            pltpu.VMEM((CHUNK,), jnp.int32),
            pltpu.VMEM((CHUNK, H), jnp.float32),
            pltpu.VMEM_SHARED((N_ACC, H), jnp.float32),
            pltpu.SemaphoreType.DMA,
        ],
    )
    def scatter_k(xls_hbm, row_hbm, col_hbm, zeros_hbm, out_hbm,
                  row_v, col_v, rows_v, acc_sh, gsem):
        cid = lax.axis_index("c")
        sid = lax.axis_index("s")
        pltpu.sync_copy(zeros_hbm, acc_sh.at[pl.ds(sid * ZROWS, ZROWS)])
        plsc.subcore_barrier()
        cnt = lax.select(cid == 0, cpt0, cpt1)
        base = lax.select(cid == 0, sid * cpt0, NS * cpt0 + sid * cpt1)

        def body(g, carry):
            chunk = base + g
            pltpu.sync_copy(row_hbm.at[chunk], row_v)
            pltpu.sync_copy(col_hbm.at[chunk], col_v)
            pltpu.async_copy(xls_hbm.at[row_v], rows_v, gsem).wait()
            pltpu.sync_copy(rows_v, acc_sh.at[col_v], add=True)
            return carry

        lax.fori_loop(0, cnt, body, 0)
        plsc.subcore_barrier()
        off = cid * N_ACC + sid * ZROWS
        pltpu.sync_copy(acc_sh.at[pl.ds(sid * ZROWS, ZROWS)],
                        out_hbm.at[pl.ds(off, ZROWS)])

    return scatter_k


def _make_degree(total_chunks, w=32):
    cpt = total_chunks // NS

    @functools.partial(
        pl.kernel,
        mesh=_mesh(),
        out_type=jax.ShapeDtypeStruct((NC * N_ACC, w), jnp.float32),
        scratch_types=[
            pltpu.VMEM((CHUNK,), jnp.int32),
            pltpu.VMEM((CHUNK, w), jnp.float32),
            pltpu.VMEM_SHARED((N_ACC, w), jnp.float32),
        ],
    )
    def degree_k(cols_hbm, ones_hbm, zeros_hbm, out_hbm, col_v, ones_v,
                 acc_sh):
        cid = lax.axis_index("c")
        sid = lax.axis_index("s")
        pltpu.sync_copy(zeros_hbm, acc_sh.at[pl.ds(sid * ZROWS, ZROWS)])
        pltpu.sync_copy(ones_hbm, ones_v)
        plsc.subcore_barrier()
        base = cid * total_chunks + sid * cpt

        def body(g, carry):
            pltpu.sync_copy(cols_hbm.at[base + g], col_v)
            pltpu.sync_copy(ones_v, acc_sh.at[col_v], add=True)
            return carry

        lax.fori_loop(0, cpt, body, 0)
        plsc.subcore_barrier()
        off = cid * N_ACC + sid * ZROWS
        pltpu.sync_copy(acc_sh.at[pl.ds(sid * ZROWS, ZROWS)],
                        out_hbm.at[pl.ds(off, ZROWS)])

    return degree_k


def _row_spec():
    return pl.BlockSpec((BR, H), lambda i: (i, 0))


def _fix_spec(shape):
    return pl.BlockSpec(shape, lambda i: tuple(0 for _ in shape))


def _mm_bias_body(x_ref, w_ref, b_ref, o_ref):
    o_ref[...] = (
        jnp.dot(x_ref[...], w_ref[...], preferred_element_type=jnp.float32)
        + b_ref[...]
    )


def _mm_bias(x, w, b):
    return pl.pallas_call(
        _mm_bias_body,
        grid=(N // BR,),
        in_specs=[_row_spec(), _fix_spec((H, H)), _fix_spec((1, H))],
        out_specs=_row_spec(),
        out_shape=jax.ShapeDtypeStruct((N, H), jnp.float32),
    )(x, w, b.reshape(1, H))


def _mm_scale_body(x_ref, w_ref, d_ref, o_ref):
    o_ref[...] = (
        jnp.dot(x_ref[...], w_ref[...], preferred_element_type=jnp.float32)
        * d_ref[...]
    )


def _mm_scale(x, w, dinv):
    return pl.pallas_call(
        _mm_scale_body,
        grid=(N // BR,),
        in_specs=[_row_spec(), _fix_spec((H, H)), _row_spec()],
        out_specs=_row_spec(),
        out_shape=jax.ShapeDtypeStruct((N, H), jnp.float32),
    )(x, w, dinv)


def _epi_body(h_ref, xls_ref, a0_ref, a1_ref, d_ref, bl_ref, g_ref, be_ref, o_ref):
    s = d_ref[...] * (a0_ref[...] + a1_ref[...] + xls_ref[...]) + bl_ref[...]
    t = h_ref[...] + s
    m = jnp.mean(t, axis=-1, keepdims=True)
    v = jnp.mean((t - m) ** 2, axis=-1, keepdims=True)
    t = (t - m) * lax.rsqrt(v + 1e-5) * g_ref[...] + be_ref[...]
    o_ref[...] = jnp.maximum(t, 0.0)


def _epi(h, xls, a0, a1, dinv, bl, g, be):
    return pl.pallas_call(
        _epi_body,
        grid=(N // BR,),
        in_specs=[_row_spec(), _row_spec(), _row_spec(), _row_spec(),
                  _row_spec(), _fix_spec((1, H)), _fix_spec((1, H)),
                  _fix_spec((1, H))],
        out_specs=_row_spec(),
        out_shape=jax.ShapeDtypeStruct((N, H), jnp.float32),
    )(h, xls, a0, a1, dinv, bl.reshape(1, H), g.reshape(1, H), be.reshape(1, H))


def _prep_edges(ei, total_chunks):
    e = ei.shape[1]
    pad = total_chunks * CHUNK - e
    row = jnp.concatenate(
        [ei[0].astype(jnp.int32), jnp.zeros((pad,), jnp.int32)]
    ).reshape(total_chunks, CHUNK)
    col = jnp.concatenate(
        [ei[1].astype(jnp.int32), jnp.full((pad,), N, jnp.int32)]
    ).reshape(total_chunks, CHUNK)
    return row, col


def _gnn(x_in, rows, cols, p, g, dinv, scatter_k):
    h = _mm_bias(x_in, p[g + "_in_W"], p[g + "_in_b"])
    for l in range(2):
        xls = _mm_scale(h, p[g + "_gcn_W"][l], dinv)
        acc = scatter_k(xls, rows, cols, jnp.zeros((ZROWS, H), jnp.float32))
        h = _epi(h, xls, acc[:N], acc[N_ACC:N_ACC + N], dinv,
                 p[g + "_gcn_b"][l], p[g + "_ln_g"], p[g + "_ln_b"])
    return _mm_bias(h, p[g + "_out_W"], p[g + "_out_b"])


def kernel(x_asset, x_index, edge_index_aa, edge_index_ai, params):
    del x_index
    p = params
    e = edge_index_aa.shape[1]
    total_chunks = -(-e // (CHUNK * NC * NS)) * (NC * NS)

    row_aa, col_aa = _prep_edges(edge_index_aa, total_chunks)
    row_ai, col_ai = _prep_edges(edge_index_ai, total_chunks)

    dw = 128
    degree_k = _make_degree(total_chunks, dw)
    scatter_k = _make_scatter(total_chunks, 66)

    deg = degree_k(
        jnp.concatenate([col_aa, col_ai], axis=0),
        jnp.ones((CHUNK, dw), jnp.float32),
        jnp.zeros((ZROWS, dw), jnp.float32),
    )
    dinv_all = lax.rsqrt(deg[:, :1] + 1.0)
    dinv_aa = jnp.broadcast_to(dinv_all[:N], (N, H))
    dinv_ai = jnp.broadcast_to(dinv_all[N_ACC:N_ACC + N], (N, H))

    h0 = _mm_bias(x_asset, p["proj_asset_W"], p["proj_asset_b"])
    h_a = _gnn(h0, row_aa, col_aa, p, "aa", dinv_aa, scatter_k)
    h_i = _gnn(h_a, row_ai, col_ai, p, "ai", dinv_ai, scatter_k)
    out_a = _mm_bias(h_a, p["out_asset_W"], p["out_asset_b"])
    out_i = _mm_bias(h_i, p["out_index_W"], p["out_index_b"])
    return (out_a, out_i)


# speedup vs baseline: 1.1428x; 1.1428x over previous
import functools

import jax
import jax.numpy as jnp
from jax import lax
from jax.experimental import pallas as pl
from jax.experimental.pallas import tpu as pltpu
from jax.experimental.pallas import tpu_sc as plsc

N = 10000
H = 128
NC = 2
NS = 16
CHUNK = 128
N_ACC = 10240
ZROWS = N_ACC // NS
BR = 2000


def _mesh():
    return plsc.VectorSubcoreMesh(
        core_axis_name="c", subcore_axis_name="s", num_cores=NC, num_subcores=NS
    )


def _make_scatter(total_chunks, cpt0):
    cpt1 = total_chunks // NS - cpt0

    @functools.partial(
        pl.kernel,
        mesh=_mesh(),
        out_type=jax.ShapeDtypeStruct((NC * N_ACC, H), jnp.float32),
        scratch_types=[
            pltpu.VMEM((CHUNK,), jnp.int32),
            pltpu.VMEM((CHUNK,), jnp.int32),
            pltpu.VMEM((CHUNK, H), jnp.float32),
            pltpu.VMEM_SHARED((N_ACC, H), jnp.float32),
            pltpu.SemaphoreType.DMA,
        ],
    )
    def scatter_k(xls_hbm, row_hbm, col_hbm, zeros_hbm, out_hbm,
                  row_v, col_v, rows_v, acc_sh, gsem):
        cid = lax.axis_index("c")
        sid = lax.axis_index("s")
        pltpu.sync_copy(zeros_hbm, acc_sh.at[pl.ds(sid * ZROWS, ZROWS)])
        plsc.subcore_barrier()
        cnt = lax.select(cid == 0, cpt0, cpt1)
        base = lax.select(cid == 0, sid * cpt0, NS * cpt0 + sid * cpt1)

        def body(g, carry):
            chunk = base + g
            pltpu.sync_copy(row_hbm.at[chunk], row_v)
            pltpu.sync_copy(col_hbm.at[chunk], col_v)
            pltpu.async_copy(xls_hbm.at[row_v], rows_v, gsem).wait()
            pltpu.sync_copy(rows_v, acc_sh.at[col_v], add=True)
            return carry

        lax.fori_loop(0, cnt, body, 0)
        plsc.subcore_barrier()
        off = cid * N_ACC + sid * ZROWS
        pltpu.sync_copy(acc_sh.at[pl.ds(sid * ZROWS, ZROWS)],
                        out_hbm.at[pl.ds(off, ZROWS)])

    return scatter_k


def _make_degree(total_chunks, w=32):
    cpt = total_chunks // NS

    @functools.partial(
        pl.kernel,
        mesh=_mesh(),
        out_type=jax.ShapeDtypeStruct((NC * N_ACC, w), jnp.float32),
        scratch_types=[
            pltpu.VMEM((CHUNK,), jnp.int32),
            pltpu.VMEM((CHUNK, w), jnp.float32),
            pltpu.VMEM_SHARED((N_ACC, w), jnp.float32),
        ],
    )
    def degree_k(cols_hbm, ones_hbm, zeros_hbm, out_hbm, col_v, ones_v,
                 acc_sh):
        cid = lax.axis_index("c")
        sid = lax.axis_index("s")
        pltpu.sync_copy(zeros_hbm, acc_sh.at[pl.ds(sid * ZROWS, ZROWS)])
        pltpu.sync_copy(ones_hbm, ones_v)
        plsc.subcore_barrier()
        base = cid * total_chunks + sid * cpt

        def body(g, carry):
            pltpu.sync_copy(cols_hbm.at[base + g], col_v)
            pltpu.sync_copy(ones_v, acc_sh.at[col_v], add=True)
            return carry

        lax.fori_loop(0, cpt, body, 0)
        plsc.subcore_barrier()
        off = cid * N_ACC + sid * ZROWS
        pltpu.sync_copy(acc_sh.at[pl.ds(sid * ZROWS, ZROWS)],
                        out_hbm.at[pl.ds(off, ZROWS)])

    return degree_k


def _row_spec():
    return pl.BlockSpec((BR, H), lambda i: (i, 0))


def _fix_spec(shape):
    return pl.BlockSpec(shape, lambda i: tuple(0 for _ in shape))


def _mm_bias_body(x_ref, w_ref, b_ref, o_ref):
    o_ref[...] = (
        jnp.dot(x_ref[...], w_ref[...], preferred_element_type=jnp.float32)
        + b_ref[...]
    )


def _mm_bias(x, w, b):
    return pl.pallas_call(
        _mm_bias_body,
        grid=(N // BR,),
        in_specs=[_row_spec(), _fix_spec((H, H)), _fix_spec((1, H))],
        out_specs=_row_spec(),
        out_shape=jax.ShapeDtypeStruct((N, H), jnp.float32),
    )(x, w, b.reshape(1, H))


def _mm_scale_body(x_ref, w_ref, d_ref, o_ref):
    o_ref[...] = (
        jnp.dot(x_ref[...], w_ref[...], preferred_element_type=jnp.float32)
        * d_ref[...]
    )


def _mm_scale(x, w, dinv):
    return pl.pallas_call(
        _mm_scale_body,
        grid=(N // BR,),
        in_specs=[_row_spec(), _fix_spec((H, H)), _row_spec()],
        out_specs=_row_spec(),
        out_shape=jax.ShapeDtypeStruct((N, H), jnp.float32),
    )(x, w, dinv)


def _epi_body(h_ref, xls_ref, a0_ref, a1_ref, d_ref, bl_ref, g_ref, be_ref, o_ref):
    s = d_ref[...] * (a0_ref[...] + a1_ref[...] + xls_ref[...]) + bl_ref[...]
    t = h_ref[...] + s
    m = jnp.mean(t, axis=-1, keepdims=True)
    v = jnp.mean((t - m) ** 2, axis=-1, keepdims=True)
    t = (t - m) * lax.rsqrt(v + 1e-5) * g_ref[...] + be_ref[...]
    o_ref[...] = jnp.maximum(t, 0.0)


def _epi(h, xls, a0, a1, dinv, bl, g, be):
    return pl.pallas_call(
        _epi_body,
        grid=(N // BR,),
        in_specs=[_row_spec(), _row_spec(), _row_spec(), _row_spec(),
                  _row_spec(), _fix_spec((1, H)), _fix_spec((1, H)),
                  _fix_spec((1, H))],
        out_specs=_row_spec(),
        out_shape=jax.ShapeDtypeStruct((N, H), jnp.float32),
    )(h, xls, a0, a1, dinv, bl.reshape(1, H), g.reshape(1, H), be.reshape(1, H))


def _prep_edges(ei, total_chunks):
    e = ei.shape[1]
    pad = total_chunks * CHUNK - e
    row = jnp.concatenate(
        [ei[0].astype(jnp.int32), jnp.zeros((pad,), jnp.int32)]
    ).reshape(total_chunks, CHUNK)
    col = jnp.concatenate(
        [ei[1].astype(jnp.int32), jnp.full((pad,), N, jnp.int32)]
    ).reshape(total_chunks, CHUNK)
    return row, col


def _gnn(x_in, rows, cols, p, g, dinv, scatter_k):
    h = _mm_bias(x_in, p[g + "_in_W"], p[g + "_in_b"])
    for l in range(2):
        xls = _mm_scale(h, p[g + "_gcn_W"][l], dinv)
        acc = scatter_k(xls, rows, cols, jnp.zeros((ZROWS, H), jnp.float32))
        h = _epi(h, xls, acc[:N], acc[N_ACC:N_ACC + N], dinv,
                 p[g + "_gcn_b"][l], p[g + "_ln_g"], p[g + "_ln_b"])
    return _mm_bias(h, p[g + "_out_W"], p[g + "_out_b"])


def kernel(x_asset, x_index, edge_index_aa, edge_index_ai, params):
    del x_index
    p = params
    e = edge_index_aa.shape[1]
    total_chunks = -(-e // (CHUNK * NC * NS)) * (NC * NS)

    row_aa, col_aa = _prep_edges(edge_index_aa, total_chunks)
    row_ai, col_ai = _prep_edges(edge_index_ai, total_chunks)

    dw = 128
    degree_k = _make_degree(total_chunks, dw)
    scatter_k = _make_scatter(total_chunks, 92)

    deg = degree_k(
        jnp.concatenate([col_aa, col_ai], axis=0),
        jnp.ones((CHUNK, dw), jnp.float32),
        jnp.zeros((ZROWS, dw), jnp.float32),
    )
    dinv_all = lax.rsqrt(deg[:, :1] + 1.0)
    dinv_aa = jnp.broadcast_to(dinv_all[:N], (N, H))
    dinv_ai = jnp.broadcast_to(dinv_all[N_ACC:N_ACC + N], (N, H))

    h0 = _mm_bias(x_asset, p["proj_asset_W"], p["proj_asset_b"])
    h_a = _gnn(h0, row_aa, col_aa, p, "aa", dinv_aa, scatter_k)
    h_i = _gnn(h_a, row_ai, col_ai, p, "ai", dinv_ai, scatter_k)
    out_a = _mm_bias(h_a, p["out_asset_W"], p["out_asset_b"])
    out_i = _mm_bias(h_i, p["out_index_W"], p["out_index_b"])
    return (out_a, out_i)


# speedup vs baseline: 1.1871x; 1.0387x over previous
import functools

import jax
import jax.numpy as jnp
from jax import lax
from jax.experimental import pallas as pl
from jax.experimental.pallas import tpu as pltpu
from jax.experimental.pallas import tpu_sc as plsc

N = 10000
H = 128
NC = 2
NS = 16
CHUNK = 128
N_ACC = 10240
ZROWS = N_ACC // NS
BR = 2000


def _mesh():
    return plsc.VectorSubcoreMesh(
        core_axis_name="c", subcore_axis_name="s", num_cores=NC, num_subcores=NS
    )


def _make_scatter(total_chunks, cpt0):
    cpt1 = total_chunks // NS - cpt0

    @functools.partial(
        pl.kernel,
        mesh=_mesh(),
        out_type=jax.ShapeDtypeStruct((NC * N_ACC, H), jnp.float32),
        scratch_types=[
            pltpu.VMEM((CHUNK,), jnp.int32),
            pltpu.VMEM((CHUNK,), jnp.int32),
            pltpu.VMEM((CHUNK, H), jnp.float32),
            pltpu.VMEM_SHARED((N_ACC, H), jnp.float32),
            pltpu.SemaphoreType.DMA,
        ],
    )
    def scatter_k(xls_hbm, row_hbm, col_hbm, zeros_hbm, out_hbm,
                  row_v, col_v, rows_v, acc_sh, gsem):
        cid = lax.axis_index("c")
        sid = lax.axis_index("s")
        pltpu.sync_copy(zeros_hbm, acc_sh.at[pl.ds(sid * ZROWS, ZROWS)])
        plsc.subcore_barrier()
        cnt = lax.select(cid == 0, cpt0, cpt1)
        base = lax.select(cid == 0, sid * cpt0, NS * cpt0 + sid * cpt1)

        def body(g, carry):
            chunk = base + g
            pltpu.sync_copy(row_hbm.at[chunk], row_v)
            pltpu.sync_copy(col_hbm.at[chunk], col_v)
            pltpu.async_copy(xls_hbm.at[row_v], rows_v, gsem).wait()
            pltpu.sync_copy(rows_v, acc_sh.at[col_v], add=True)
            return carry

        lax.fori_loop(0, cnt, body, 0)
        plsc.subcore_barrier()
        off = cid * N_ACC + sid * ZROWS
        pltpu.sync_copy(acc_sh.at[pl.ds(sid * ZROWS, ZROWS)],
                        out_hbm.at[pl.ds(off, ZROWS)])

    return scatter_k


def _make_degree(total_chunks, w=32):
    cpt = total_chunks // NS

    @functools.partial(
        pl.kernel,
        mesh=_mesh(),
        out_type=jax.ShapeDtypeStruct((NC * N_ACC, w), jnp.float32),
        scratch_types=[
            pltpu.VMEM((CHUNK,), jnp.int32),
            pltpu.VMEM((CHUNK, w), jnp.float32),
            pltpu.VMEM_SHARED((N_ACC, w), jnp.float32),
        ],
    )
    def degree_k(cols_hbm, ones_hbm, zeros_hbm, out_hbm, col_v, ones_v,
                 acc_sh):
        cid = lax.axis_index("c")
        sid = lax.axis_index("s")
        pltpu.sync_copy(zeros_hbm, acc_sh.at[pl.ds(sid * ZROWS, ZROWS)])
        pltpu.sync_copy(ones_hbm, ones_v)
        plsc.subcore_barrier()
        base = cid * total_chunks + sid * cpt

        def body(g, carry):
            pltpu.sync_copy(cols_hbm.at[base + g], col_v)
            pltpu.sync_copy(ones_v, acc_sh.at[col_v], add=True)
            return carry

        lax.fori_loop(0, cpt, body, 0)
        plsc.subcore_barrier()
        off = cid * N_ACC + sid * ZROWS
        pltpu.sync_copy(acc_sh.at[pl.ds(sid * ZROWS, ZROWS)],
                        out_hbm.at[pl.ds(off, ZROWS)])

    return degree_k


def _row_spec():
    return pl.BlockSpec((BR, H), lambda i: (i, 0))


def _fix_spec(shape):
    return pl.BlockSpec(shape, lambda i: tuple(0 for _ in shape))


def _mm_bias_body(x_ref, w_ref, b_ref, o_ref):
    o_ref[...] = (
        jnp.dot(x_ref[...], w_ref[...], preferred_element_type=jnp.float32)
        + b_ref[...]
    )


def _mm_bias(x, w, b):
    return pl.pallas_call(
        _mm_bias_body,
        grid=(N // BR,),
        in_specs=[_row_spec(), _fix_spec((H, H)), _fix_spec((1, H))],
        out_specs=_row_spec(),
        out_shape=jax.ShapeDtypeStruct((N, H), jnp.float32),
    )(x, w, b.reshape(1, H))


def _mm_scale_body(x_ref, w_ref, d_ref, o_ref):
    o_ref[...] = (
        jnp.dot(x_ref[...], w_ref[...], preferred_element_type=jnp.float32)
        * d_ref[...]
    )


def _mm_scale(x, w, dinv):
    return pl.pallas_call(
        _mm_scale_body,
        grid=(N // BR,),
        in_specs=[_row_spec(), _fix_spec((H, H)), _row_spec()],
        out_specs=_row_spec(),
        out_shape=jax.ShapeDtypeStruct((N, H), jnp.float32),
    )(x, w, dinv)


def _epi_body(h_ref, xls_ref, a0_ref, a1_ref, d_ref, bl_ref, g_ref, be_ref, o_ref):
    s = d_ref[...] * (a0_ref[...] + a1_ref[...] + xls_ref[...]) + bl_ref[...]
    t = h_ref[...] + s
    m = jnp.mean(t, axis=-1, keepdims=True)
    v = jnp.mean((t - m) ** 2, axis=-1, keepdims=True)
    t = (t - m) * lax.rsqrt(v + 1e-5) * g_ref[...] + be_ref[...]
    o_ref[...] = jnp.maximum(t, 0.0)


def _epi(h, xls, a0, a1, dinv, bl, g, be):
    return pl.pallas_call(
        _epi_body,
        grid=(N // BR,),
        in_specs=[_row_spec(), _row_spec(), _row_spec(), _row_spec(),
                  _row_spec(), _fix_spec((1, H)), _fix_spec((1, H)),
                  _fix_spec((1, H))],
        out_specs=_row_spec(),
        out_shape=jax.ShapeDtypeStruct((N, H), jnp.float32),
    )(h, xls, a0, a1, dinv, bl.reshape(1, H), g.reshape(1, H), be.reshape(1, H))


def _prep_edges(ei, total_chunks):
    e = ei.shape[1]
    pad = total_chunks * CHUNK - e
    row = jnp.concatenate(
        [ei[0].astype(jnp.int32), jnp.zeros((pad,), jnp.int32)]
    ).reshape(total_chunks, CHUNK)
    col = jnp.concatenate(
        [ei[1].astype(jnp.int32), jnp.full((pad,), N, jnp.int32)]
    ).reshape(total_chunks, CHUNK)
    return row, col


def _gnn(x_in, rows, cols, p, g, dinv, scatter_k):
    h = _mm_bias(x_in, p[g + "_in_W"], p[g + "_in_b"])
    for l in range(2):
        xls = _mm_scale(h, p[g + "_gcn_W"][l], dinv)
        acc = scatter_k(xls, rows, cols, jnp.zeros((ZROWS, H), jnp.float32))
        h = _epi(h, xls, acc[:N], acc[N_ACC:N_ACC + N], dinv,
                 p[g + "_gcn_b"][l], p[g + "_ln_g"], p[g + "_ln_b"])
    return _mm_bias(h, p[g + "_out_W"], p[g + "_out_b"])


def kernel(x_asset, x_index, edge_index_aa, edge_index_ai, params):
    del x_index
    p = params
    e = edge_index_aa.shape[1]
    total_chunks = -(-e // (CHUNK * NC * NS)) * (NC * NS)

    row_aa, col_aa = _prep_edges(edge_index_aa, total_chunks)
    row_ai, col_ai = _prep_edges(edge_index_ai, total_chunks)

    dw = 128
    degree_k = _make_degree(total_chunks, dw)
    scatter_k = _make_scatter(total_chunks, 98)

    deg = degree_k(
        jnp.concatenate([col_aa, col_ai], axis=0),
        jnp.ones((CHUNK, dw), jnp.float32),
        jnp.zeros((ZROWS, dw), jnp.float32),
    )
    dinv_all = lax.rsqrt(deg[:, :1] + 1.0)
    dinv_aa = jnp.broadcast_to(dinv_all[:N], (N, H))
    dinv_ai = jnp.broadcast_to(dinv_all[N_ACC:N_ACC + N], (N, H))

    h0 = _mm_bias(x_asset, p["proj_asset_W"], p["proj_asset_b"])
    h_a = _gnn(h0, row_aa, col_aa, p, "aa", dinv_aa, scatter_k)
    h_i = _gnn(h_a, row_ai, col_ai, p, "ai", dinv_ai, scatter_k)
    out_a = _mm_bias(h_a, p["out_asset_W"], p["out_asset_b"])
    out_i = _mm_bias(h_i, p["out_index_W"], p["out_index_b"])
    return (out_a, out_i)


# speedup vs baseline: 1.2229x; 1.0302x over previous
import functools

import jax
import jax.numpy as jnp
from jax import lax
from jax.experimental import pallas as pl
from jax.experimental.pallas import tpu as pltpu
from jax.experimental.pallas import tpu_sc as plsc

N = 10000
H = 128
NC = 2
NS = 16
CHUNK = 128
N_ACC = 10240
ZROWS = N_ACC // NS
BR = 2000


def _mesh():
    return plsc.VectorSubcoreMesh(
        core_axis_name="c", subcore_axis_name="s", num_cores=NC, num_subcores=NS
    )


def _make_scatter(total_chunks, cpt0):
    cpt1 = total_chunks // NS - cpt0

    @functools.partial(
        pl.kernel,
        mesh=_mesh(),
        out_type=jax.ShapeDtypeStruct((NC * N_ACC, H), jnp.float32),
        scratch_types=[
            pltpu.VMEM((CHUNK,), jnp.int32),
            pltpu.VMEM((CHUNK,), jnp.int32),
            pltpu.VMEM((CHUNK, H), jnp.float32),
            pltpu.VMEM_SHARED((N_ACC, H), jnp.float32),
            pltpu.SemaphoreType.DMA,
        ],
    )
    def scatter_k(xls_hbm, row_hbm, col_hbm, zeros_hbm, out_hbm,
                  row_v, col_v, rows_v, acc_sh, gsem):
        cid = lax.axis_index("c")
        sid = lax.axis_index("s")
        pltpu.sync_copy(zeros_hbm, acc_sh.at[pl.ds(sid * ZROWS, ZROWS)])
        plsc.subcore_barrier()
        cnt = lax.select(cid == 0, cpt0, cpt1)
        base = lax.select(cid == 0, sid * cpt0, NS * cpt0 + sid * cpt1)

        def body(g, carry):
            chunk = base + g
            pltpu.sync_copy(row_hbm.at[chunk], row_v)
            pltpu.sync_copy(col_hbm.at[chunk], col_v)
            pltpu.async_copy(xls_hbm.at[row_v], rows_v, gsem).wait()
            pltpu.sync_copy(rows_v, acc_sh.at[col_v], add=True)
            return carry

        lax.fori_loop(0, cnt, body, 0)
        plsc.subcore_barrier()
        off = cid * N_ACC + sid * ZROWS
        pltpu.sync_copy(acc_sh.at[pl.ds(sid * ZROWS, ZROWS)],
                        out_hbm.at[pl.ds(off, ZROWS)])

    return scatter_k


def _make_degree(total_chunks, w=32):
    cpt = total_chunks // NS

    @functools.partial(
        pl.kernel,
        mesh=_mesh(),
        out_type=jax.ShapeDtypeStruct((NC * N_ACC, w), jnp.float32),
        scratch_types=[
            pltpu.VMEM((CHUNK,), jnp.int32),
            pltpu.VMEM((CHUNK, w), jnp.float32),
            pltpu.VMEM_SHARED((N_ACC, w), jnp.float32),
        ],
    )
    def degree_k(cols_hbm, ones_hbm, zeros_hbm, out_hbm, col_v, ones_v,
                 acc_sh):
        cid = lax.axis_index("c")
        sid = lax.axis_index("s")
        pltpu.sync_copy(zeros_hbm, acc_sh.at[pl.ds(sid * ZROWS, ZROWS)])
        pltpu.sync_copy(ones_hbm, ones_v)
        plsc.subcore_barrier()
        base = cid * total_chunks + sid * cpt

        def body(g, carry):
            pltpu.sync_copy(cols_hbm.at[base + g], col_v)
            pltpu.sync_copy(ones_v, acc_sh.at[col_v], add=True)
            return carry

        lax.fori_loop(0, cpt, body, 0)
        plsc.subcore_barrier()
        off = cid * N_ACC + sid * ZROWS
        pltpu.sync_copy(acc_sh.at[pl.ds(sid * ZROWS, ZROWS)],
                        out_hbm.at[pl.ds(off, ZROWS)])

    return degree_k


def _row_spec():
    return pl.BlockSpec((BR, H), lambda i: (i, 0))


def _fix_spec(shape):
    return pl.BlockSpec(shape, lambda i: tuple(0 for _ in shape))


def _mm_bias_body(x_ref, w_ref, b_ref, o_ref):
    o_ref[...] = (
        jnp.dot(x_ref[...], w_ref[...], preferred_element_type=jnp.float32)
        + b_ref[...]
    )


def _mm_bias(x, w, b):
    return pl.pallas_call(
        _mm_bias_body,
        grid=(N // BR,),
        in_specs=[_row_spec(), _fix_spec((H, H)), _fix_spec((1, H))],
        out_specs=_row_spec(),
        out_shape=jax.ShapeDtypeStruct((N, H), jnp.float32),
    )(x, w, b.reshape(1, H))


def _mm_scale_body(x_ref, w_ref, d_ref, o_ref):
    o_ref[...] = (
        jnp.dot(x_ref[...], w_ref[...], preferred_element_type=jnp.float32)
        * d_ref[...]
    )


def _mm_scale(x, w, dinv):
    return pl.pallas_call(
        _mm_scale_body,
        grid=(N // BR,),
        in_specs=[_row_spec(), _fix_spec((H, H)), _row_spec()],
        out_specs=_row_spec(),
        out_shape=jax.ShapeDtypeStruct((N, H), jnp.float32),
    )(x, w, dinv)


def _epi_body(h_ref, xls_ref, a0_ref, a1_ref, d_ref, bl_ref, g_ref, be_ref, o_ref):
    s = d_ref[...] * (a0_ref[...] + a1_ref[...] + xls_ref[...]) + bl_ref[...]
    t = h_ref[...] + s
    m = jnp.mean(t, axis=-1, keepdims=True)
    v = jnp.mean((t - m) ** 2, axis=-1, keepdims=True)
    t = (t - m) * lax.rsqrt(v + 1e-5) * g_ref[...] + be_ref[...]
    o_ref[...] = jnp.maximum(t, 0.0)


def _epi(h, xls, a0, a1, dinv, bl, g, be):
    return pl.pallas_call(
        _epi_body,
        grid=(N // BR,),
        in_specs=[_row_spec(), _row_spec(), _row_spec(), _row_spec(),
                  _row_spec(), _fix_spec((1, H)), _fix_spec((1, H)),
                  _fix_spec((1, H))],
        out_specs=_row_spec(),
        out_shape=jax.ShapeDtypeStruct((N, H), jnp.float32),
    )(h, xls, a0, a1, dinv, bl.reshape(1, H), g.reshape(1, H), be.reshape(1, H))


def _prep_edges(ei, total_chunks):
    e = ei.shape[1]
    pad = total_chunks * CHUNK - e
    row = jnp.concatenate(
        [ei[0].astype(jnp.int32), jnp.zeros((pad,), jnp.int32)]
    ).reshape(total_chunks, CHUNK)
    col = jnp.concatenate(
        [ei[1].astype(jnp.int32), jnp.full((pad,), N, jnp.int32)]
    ).reshape(total_chunks, CHUNK)
    return row, col


def _gnn(x_in, rows, cols, p, g, dinv, scatter_k):
    h = _mm_bias(x_in, p[g + "_in_W"], p[g + "_in_b"])
    for l in range(2):
        xls = _mm_scale(h, p[g + "_gcn_W"][l], dinv)
        acc = scatter_k(xls, rows, cols, jnp.zeros((ZROWS, H), jnp.float32))
        h = _epi(h, xls, acc[:N], acc[N_ACC:N_ACC + N], dinv,
                 p[g + "_gcn_b"][l], p[g + "_ln_g"], p[g + "_ln_b"])
    return _mm_bias(h, p[g + "_out_W"], p[g + "_out_b"])


def kernel(x_asset, x_index, edge_index_aa, edge_index_ai, params):
    del x_index
    p = params
    e = edge_index_aa.shape[1]
    total_chunks = -(-e // (CHUNK * NC * NS)) * (NC * NS)

    row_aa, col_aa = _prep_edges(edge_index_aa, total_chunks)
    row_ai, col_ai = _prep_edges(edge_index_ai, total_chunks)

    dw = 128
    degree_k = _make_degree(total_chunks, dw)
    scatter_k = _make_scatter(total_chunks, 104)

    deg = degree_k(
        jnp.concatenate([col_aa, col_ai], axis=0),
        jnp.ones((CHUNK, dw), jnp.float32),
        jnp.zeros((ZROWS, dw), jnp.float32),
    )
    dinv_all = lax.rsqrt(deg[:, :1] + 1.0)
    dinv_aa = jnp.broadcast_to(dinv_all[:N], (N, H))
    dinv_ai = jnp.broadcast_to(dinv_all[N_ACC:N_ACC + N], (N, H))

    h0 = _mm_bias(x_asset, p["proj_asset_W"], p["proj_asset_b"])
    h_a = _gnn(h0, row_aa, col_aa, p, "aa", dinv_aa, scatter_k)
    h_i = _gnn(h_a, row_ai, col_ai, p, "ai", dinv_ai, scatter_k)
    out_a = _mm_bias(h_a, p["out_asset_W"], p["out_asset_b"])
    out_i = _mm_bias(h_i, p["out_index_W"], p["out_index_b"])
    return (out_a, out_i)
